# baseline (speedup 1.0000x reference)
import jax
import jax.numpy as jnp
from jax import lax
from jax.experimental import pallas as pl
from jax.experimental.pallas import tpu as pltpu
from jax.experimental.pallas import tpu_sc as plsc

_NCODES = 1024
_HW = 1024
_D = 64
_NIMG = 16
_NELEM = float(_NIMG * _HW * _D)
_NW = 32
_PPW = (_NIMG * _HW) // _NW
_LANES = 16


def _vq_body(z_ref, en_ref, embm2_ref, idx_ref, loss_ref):
    n = pl.program_id(0)
    z2d = z_ref[0]
    zn = jnp.sum(z2d * z2d, axis=0, keepdims=True)
    en = en_ref[...]
    dotm2 = lax.dot_general(
        embm2_ref[...], z2d, (((1,), (0,)), ((), ())),
        preferred_element_type=jnp.float32,
    )
    dist = (zn + en) + dotm2
    m = jnp.min(dist, axis=0, keepdims=True)
    kio = lax.broadcasted_iota(jnp.int32, (_NCODES, _HW), 0)
    idx = jnp.min(jnp.where(dist == m, kio, _NCODES), axis=0, keepdims=True)
    idx_ref[0] = idx
    part = jnp.sum(m, keepdims=True).reshape(1, 1)
    prev = jnp.where(n == 0, jnp.zeros((1, 1), jnp.float32), loss_ref[...])
    total = prev + part
    loss_ref[...] = jnp.where(n == _NIMG - 1, total / _NELEM, total)


def _gather_body(emb_hbm, idx_hbm, zq_hbm, emb_v, idx_v, out_v):
    wid = lax.axis_index("s") * 2 + lax.axis_index("c")
    base = wid * _PPW
    img = wid // 2
    p0 = (wid % 2) * _PPW
    pltpu.sync_copy(emb_hbm, emb_v)
    pltpu.sync_copy(idx_hbm.at[pl.ds(base, _PPW)], idx_v)

    def body(g, carry):
        s = g * _LANES
        idxv = idx_v[pl.ds(s, _LANES)]
        flat0 = idxv * _D
        for c in range(_D):
            out_v[c, pl.ds(s, _LANES)] = plsc.load_gather(emb_v, [flat0 + c])
        return carry

    lax.fori_loop(0, _PPW // _LANES, body, 0)
    pltpu.sync_copy(out_v, zq_hbm.at[img, :, pl.ds(p0, _PPW)])


def _sc_gather(embedding, idx_flat):
    f = pl.kernel(
        _gather_body,
        out_type=jax.ShapeDtypeStruct((_NIMG, _D, _HW), jnp.float32),
        mesh=plsc.VectorSubcoreMesh(
            core_axis_name="c", subcore_axis_name="s",
            num_cores=2, num_subcores=16,
        ),
        scratch_types=[
            pltpu.VMEM((_NCODES * _D,), jnp.float32),
            pltpu.VMEM((_PPW,), jnp.int32),
            pltpu.VMEM((_D, _PPW), jnp.float32),
        ],
        compiler_params=pltpu.CompilerParams(needs_layout_passes=False),
    )
    return f(embedding.reshape(_NCODES * _D), idx_flat)


def kernel(z_e, embedding):
    z_r = z_e.reshape(_NIMG, _D, _HW)
    en_in = jnp.sum(embedding**2, axis=1, keepdims=True)
    embm2 = embedding * -2.0
    idx3, loss = pl.pallas_call(
        _vq_body,
        grid=(_NIMG,),
        in_specs=[
            pl.BlockSpec((1, _D, _HW), lambda n: (n, 0, 0)),
            pl.BlockSpec((_NCODES, 1), lambda n: (0, 0)),
            pl.BlockSpec((_NCODES, _D), lambda n: (0, 0)),
        ],
        out_specs=(
            pl.BlockSpec((1, 1, _HW), lambda n: (n, 0, 0)),
            pl.BlockSpec((1, 1), lambda n: (0, 0)),
        ),
        out_shape=(
            jax.ShapeDtypeStruct((_NIMG, 1, _HW), jnp.int32),
            jax.ShapeDtypeStruct((1, 1), jnp.float32),
        ),
    )(z_r, en_in, embm2)
    idx_flat = idx3.reshape(_NIMG * _HW)
    zq = _sc_gather(embedding, idx_flat)
    z_q = zq.reshape(_NIMG, _D, 32, 32)
    indices = idx3.reshape(_NIMG, 32, 32)
    return (z_q, loss[0, 0], indices)


# speedup vs baseline: 1.1933x; 1.1933x over previous
import jax
import jax.numpy as jnp
from jax import lax
from jax.experimental import pallas as pl
from jax.experimental.pallas import tpu as pltpu
from jax.experimental.pallas import tpu_sc as plsc

_NCODES = 1024
_HW = 1024
_D = 64
_NIMG = 16
_NELEM = float(_NIMG * _HW * _D)
_NW = 32
_PPW = (_NIMG * _HW) // _NW
_LANES = 16


def _vq_body(z_ref, en_ref, embm2_ref, idx_ref, loss_ref):
    n = pl.program_id(0)
    z2d = z_ref[0]
    zn = jnp.sum(z2d * z2d, axis=0, keepdims=True)
    en = en_ref[...]
    dotm2 = lax.dot_general(
        embm2_ref[...], z2d, (((1,), (0,)), ((), ())),
        preferred_element_type=jnp.float32,
    )
    dist = (zn + en) + dotm2
    m = jnp.min(dist, axis=0, keepdims=True)
    kio = lax.broadcasted_iota(jnp.int32, (_NCODES, _HW), 0)
    idx = jnp.min(jnp.where(dist == m, kio, _NCODES), axis=0, keepdims=True)
    idx_ref[0] = idx
    part = jnp.sum(m, keepdims=True).reshape(1, 1)
    prev = jnp.where(n == 0, jnp.zeros((1, 1), jnp.float32), loss_ref[...])
    total = prev + part
    loss_ref[...] = jnp.where(n == _NIMG - 1, total / _NELEM, total)


def _gather_body(emb_hbm, idx_hbm, zq_hbm, emb_v, idx_v, out_v):
    wid = lax.axis_index("s") * 2 + lax.axis_index("c")
    base = wid * _PPW
    img = wid // 2
    p0 = (wid % 2) * _PPW
    pltpu.sync_copy(emb_hbm, emb_v)
    pltpu.sync_copy(idx_hbm.at[pl.ds(base, _PPW)], idx_v)

    def body(g, carry):
        s = g * _LANES
        idxv = idx_v[pl.ds(s, _LANES)]
        for c in range(_D):
            out_v[c, pl.ds(s, _LANES)] = plsc.load_gather(
                emb_v, [idxv + c * _NCODES])
        return carry

    lax.fori_loop(0, _PPW // _LANES, body, 0)
    pltpu.sync_copy(out_v, zq_hbm.at[img, :, pl.ds(p0, _PPW)])


def _sc_gather(embedding, idx_flat):
    f = pl.kernel(
        _gather_body,
        out_type=jax.ShapeDtypeStruct((_NIMG, _D, _HW), jnp.float32),
        mesh=plsc.VectorSubcoreMesh(
            core_axis_name="c", subcore_axis_name="s",
            num_cores=2, num_subcores=16,
        ),
        scratch_types=[
            pltpu.VMEM((_NCODES * _D,), jnp.float32),
            pltpu.VMEM((_PPW,), jnp.int32),
            pltpu.VMEM((_D, _PPW), jnp.float32),
        ],
        compiler_params=pltpu.CompilerParams(needs_layout_passes=False),
    )
    return f(embedding.T.reshape(_D * _NCODES), idx_flat)


def kernel(z_e, embedding):
    z_r = z_e.reshape(_NIMG, _D, _HW)
    en_in = jnp.sum(embedding**2, axis=1, keepdims=True)
    embm2 = embedding * -2.0
    idx3, loss = pl.pallas_call(
        _vq_body,
        grid=(_NIMG,),
        in_specs=[
            pl.BlockSpec((1, _D, _HW), lambda n: (n, 0, 0)),
            pl.BlockSpec((_NCODES, 1), lambda n: (0, 0)),
            pl.BlockSpec((_NCODES, _D), lambda n: (0, 0)),
        ],
        out_specs=(
            pl.BlockSpec((1, 1, _HW), lambda n: (n, 0, 0)),
            pl.BlockSpec((1, 1), lambda n: (0, 0)),
        ),
        out_shape=(
            jax.ShapeDtypeStruct((_NIMG, 1, _HW), jnp.int32),
            jax.ShapeDtypeStruct((1, 1), jnp.float32),
        ),
    )(z_r, en_in, embm2)
    idx_flat = idx3.reshape(_NIMG * _HW)
    zq = _sc_gather(embedding, idx_flat)
    z_q = zq.reshape(_NIMG, _D, 32, 32)
    indices = idx3.reshape(_NIMG, 32, 32)
    return (z_q, loss[0, 0], indices)


# speedup vs baseline: 1.2938x; 1.0842x over previous
import jax
import jax.numpy as jnp
from jax import lax
from jax.experimental import pallas as pl
from jax.experimental.pallas import tpu as pltpu
from jax.experimental.pallas import tpu_sc as plsc

_NCODES = 1024
_HW = 1024
_D = 64
_NIMG = 16
_NELEM = float(_NIMG * _HW * _D)
_NW = 32
_PPW = (_NIMG * _HW) // _NW
_LANES = 16


def _vq_body(z_ref, en_ref, embm2_ref, idx_ref, loss_ref):
    n = pl.program_id(0)
    z2d = z_ref[0]
    zn = jnp.sum(z2d * z2d, axis=0, keepdims=True)
    en = en_ref[...]
    dotm2 = lax.dot_general(
        embm2_ref[...], z2d, (((1,), (0,)), ((), ())),
        preferred_element_type=jnp.float32,
    )
    dist = (zn + en) + dotm2
    m = jnp.min(dist, axis=0, keepdims=True)
    kio = lax.broadcasted_iota(jnp.int32, (_NCODES, _HW), 0)
    idx = jnp.min(jnp.where(dist == m, kio, _NCODES), axis=0, keepdims=True)
    idx_ref[0] = idx
    part = jnp.sum(m, keepdims=True).reshape(1, 1)
    prev = jnp.where(n == 0, jnp.zeros((1, 1), jnp.float32), loss_ref[...])
    total = prev + part
    loss_ref[...] = jnp.where(n == _NIMG - 1, total / _NELEM, total)


def _gather_body(emb_hbm, idx_hbm, zq_hbm, emb_v, idx_v, out_v):
    wid = lax.axis_index("s") * 2 + lax.axis_index("c")
    base = wid * _PPW
    img = wid // 2
    p0 = (wid % 2) * _PPW
    pltpu.sync_copy(emb_hbm, emb_v)
    pltpu.sync_copy(idx_hbm.at[pl.ds(base, _PPW)], idx_v)

    @plsc.parallel_loop(0, _PPW // _LANES, unroll=2)
    def _gather_loop(g):
        s = g * _LANES
        idxv = idx_v[pl.ds(s, _LANES)]
        for c in range(_D):
            out_v[c, pl.ds(s, _LANES)] = plsc.load_gather(
                emb_v, [idxv + c * _NCODES])
    pltpu.sync_copy(out_v, zq_hbm.at[img, :, pl.ds(p0, _PPW)])


def _sc_gather(embedding, idx_flat):
    f = pl.kernel(
        _gather_body,
        out_type=jax.ShapeDtypeStruct((_NIMG, _D, _HW), jnp.float32),
        mesh=plsc.VectorSubcoreMesh(
            core_axis_name="c", subcore_axis_name="s",
            num_cores=2, num_subcores=16,
        ),
        scratch_types=[
            pltpu.VMEM((_NCODES * _D,), jnp.float32),
            pltpu.VMEM((_PPW,), jnp.int32),
            pltpu.VMEM((_D, _PPW), jnp.float32),
        ],
        compiler_params=pltpu.CompilerParams(needs_layout_passes=False),
    )
    return f(embedding.T.reshape(_D * _NCODES), idx_flat)


def kernel(z_e, embedding):
    z_r = z_e.reshape(_NIMG, _D, _HW)
    en_in = jnp.sum(embedding**2, axis=1, keepdims=True)
    embm2 = embedding * -2.0
    idx3, loss = pl.pallas_call(
        _vq_body,
        grid=(_NIMG,),
        in_specs=[
            pl.BlockSpec((1, _D, _HW), lambda n: (n, 0, 0)),
            pl.BlockSpec((_NCODES, 1), lambda n: (0, 0)),
            pl.BlockSpec((_NCODES, _D), lambda n: (0, 0)),
        ],
        out_specs=(
            pl.BlockSpec((1, 1, _HW), lambda n: (n, 0, 0)),
            pl.BlockSpec((1, 1), lambda n: (0, 0)),
        ),
        out_shape=(
            jax.ShapeDtypeStruct((_NIMG, 1, _HW), jnp.int32),
            jax.ShapeDtypeStruct((1, 1), jnp.float32),
        ),
    )(z_r, en_in, embm2)
    idx_flat = idx3.reshape(_NIMG * _HW)
    zq = _sc_gather(embedding, idx_flat)
    z_q = zq.reshape(_NIMG, _D, 32, 32)
    indices = idx3.reshape(_NIMG, 32, 32)
    return (z_q, loss[0, 0], indices)
